# baseline (speedup 1.0000x reference)
import functools

import jax
import jax.numpy as jnp
from jax import lax
from jax.experimental import pallas as pl
from jax.experimental.pallas import tpu as pltpu
from jax.experimental.pallas import tpu_sc as plsc

N = 25000
NPAD = 25600
E = 150000
H = 128
R = 200
NB = NPAD // R
NBO = N // R
DUMMY_DST = N

NSUB = 16
SUBV = 128
NJ = 8
G = NJ * SUBV
NCHUNK = 10
EPT = NSUB * NCHUNK * G
HALF = NPAD // 2
ACC_ROWS = HALF + SUBV
ROWS_PER_SID = HALF // NSUB
ZROWS = 32

_sc_mesh = plsc.VectorSubcoreMesh(core_axis_name="c", subcore_axis_name="s")


def _leaky(x):
    return jnp.where(x >= 0, x, 0.01 * x)


def _pretrans_body(xop_ref, xm_ref, wm_ref, bm_ref,
                   w1_ref, w2_ref, w3_ref, w4_ref,
                   y1_ref, y2_ref, y3_ref, y4_ref):
    xop = xop_ref[...]
    y1_ref[...] = jnp.dot(xop, w1_ref[...], preferred_element_type=jnp.float32)
    y2_ref[...] = jnp.dot(xop, w2_ref[...], preferred_element_type=jnp.float32)
    xm7 = jnp.dot(xm_ref[...], wm_ref[...],
                  preferred_element_type=jnp.float32) + bm_ref[...]
    y3_ref[...] = jnp.dot(xm7, w3_ref[...], preferred_element_type=jnp.float32)
    y4_ref[...] = jnp.dot(xm7, w4_ref[...], preferred_element_type=jnp.float32)


def _pretrans(x_op_pad, x_m_pad, Wm_p, bm_p, w1, w2, w3, w4):
    out = jax.ShapeDtypeStruct((NPAD, H), jnp.float32)
    full = lambda shape: pl.BlockSpec(shape, lambda i: (0, 0))
    return pl.pallas_call(
        _pretrans_body,
        grid=(NB,),
        in_specs=[
            pl.BlockSpec((R, 16), lambda i: (i, 0)),
            pl.BlockSpec((R, 8), lambda i: (i, 0)),
            full((8, 16)), full((1, 16)),
            full((16, H)), full((16, H)), full((16, H)), full((16, H)),
        ],
        out_specs=[pl.BlockSpec((R, H), lambda i: (i, 0))] * 4,
        out_shape=[out, out, out, out],
    )(x_op_pad, x_m_pad, Wm_p, bm_p, w1, w2, w3, w4)


def _layer0_body(a1_ref, a3_ref, b1a_ref, w1b_ref, b1b_ref,
                 b3a_ref, w3b_ref, b3b_ref, out_ref):
    h1 = jnp.maximum(a1_ref[...] + b1a_ref[...], 0.0)
    t1 = jnp.dot(h1, w1b_ref[...], preferred_element_type=jnp.float32) + b1b_ref[...]
    h3 = jnp.maximum(a3_ref[...] + b3a_ref[...], 0.0)
    t3 = jnp.dot(h3, w3b_ref[...], preferred_element_type=jnp.float32) + b3b_ref[...]
    out_ref[...] = _leaky(t1 + t3)


def _layer0_half(agg0, t1, t3, b1a, w1b, b1b, b3a, w3b, b3b):
    full = lambda shape: pl.BlockSpec(shape, lambda i: (0, 0))
    return pl.pallas_call(
        _layer0_body,
        grid=(NB,),
        in_specs=[
            pl.BlockSpec((R, H), lambda i, t=t1: (i + t * NB, 0)),
            pl.BlockSpec((R, H), lambda i, t=t3: (i + t * NB, 0)),
            full((1, H)), full((H, H)), full((1, H)),
            full((1, H)), full((H, H)), full((1, H)),
        ],
        out_specs=pl.BlockSpec((R, H), lambda i: (i, 0)),
        out_shape=jax.ShapeDtypeStruct((NPAD, H), jnp.float32),
    )(agg0, agg0, b1a, w1b, b1b, b3a, w3b, b3b)


def _layer1_body(a1_ref, a3_ref,
                 w1a_ref, b1a_ref, w1b_ref, b1b_ref,
                 w3a_ref, b3a_ref, w3b_ref, b3b_ref,
                 wfc_ref, bfc_ref, out_ref):
    h1 = jnp.maximum(jnp.dot(a1_ref[...], w1a_ref[...],
                             preferred_element_type=jnp.float32) + b1a_ref[...], 0.0)
    t1 = jnp.dot(h1, w1b_ref[...], preferred_element_type=jnp.float32) + b1b_ref[...]
    h3 = jnp.maximum(jnp.dot(a3_ref[...], w3a_ref[...],
                             preferred_element_type=jnp.float32) + b3a_ref[...], 0.0)
    t3 = jnp.dot(h3, w3b_ref[...], preferred_element_type=jnp.float32) + b3b_ref[...]
    x = _leaky(t1 + t3)
    out_ref[...] = jnp.dot(x, wfc_ref[...],
                           preferred_element_type=jnp.float32) + bfc_ref[...]


def _layer1_half(agg1, t1, t3, w1a, b1a, w1b, b1b, w3a, b3a, w3b, b3b,
                 wfc, bfc):
    full = lambda shape: pl.BlockSpec(shape, lambda i: (0, 0))
    return pl.pallas_call(
        _layer1_body,
        grid=(NBO,),
        in_specs=[
            pl.BlockSpec((R, H), lambda i, t=t1: (i + t * NB, 0)),
            pl.BlockSpec((R, H), lambda i, t=t3: (i + t * NB, 0)),
            full((H, H)), full((1, H)), full((H, H)), full((1, H)),
            full((H, H)), full((1, H)), full((H, H)), full((1, H)),
            full((H, H)), full((1, H)),
        ],
        out_specs=pl.BlockSpec((R, H), lambda i: (i, 0)),
        out_shape=jax.ShapeDtypeStruct((N, H), jnp.float32),
    )(agg1, agg1, w1a, b1a, w1b, b1b, w3a, b3a, w3b, b3b, wfc, bfc)


def _sc_agg_body(xs0, xs1, xs2, xs3, src, dst, out,
                 acc, sidx, didx, didx2, gbuf, zbuf, sem):
    c = lax.axis_index("c")
    sid = lax.axis_index("s")
    base_c = c * HALF

    z16 = jnp.zeros((16,), jnp.float32)

    @pl.loop(0, ZROWS)
    def _(i):
        for q in range(H // 16):
            zbuf[i, pl.ds(q * 16, 16)] = z16

    for t, xsrc in enumerate((xs0, xs1, xs2, xs3)):
        @pl.loop(0, ROWS_PER_SID // ZROWS)
        def _(j):
            pltpu.sync_copy(
                zbuf, acc.at[pl.ds(sid * ROWS_PER_SID + j * ZROWS, ZROWS)])
        plsc.subcore_barrier()

        @pl.loop(0, NCHUNK)
        def _(k, xsrc=xsrc, t=t):
            row0 = ((t * NSUB + sid) * NCHUNK + k) * NJ
            pltpu.sync_copy(src.at[pl.ds(row0, NJ)], sidx)
            pltpu.sync_copy(dst.at[pl.ds(row0, NJ)], didx)
            for j in range(NJ):
                for q in range(SUBV // 16):
                    d = didx[j, pl.ds(q * 16, 16)]
                    dl = d - base_c
                    ok = (dl >= 0) & (dl < HALF)
                    didx2[j, pl.ds(q * 16, 16)] = jnp.where(
                        ok, dl, HALF + (d & (SUBV - 1)))
                pltpu.async_copy(xsrc.at[sidx.at[j]], gbuf, sem).wait()
                pltpu.sync_copy(gbuf, acc.at[didx2.at[j]], add=True)

        plsc.subcore_barrier()
        pltpu.sync_copy(
            acc.at[pl.ds(sid * ROWS_PER_SID, ROWS_PER_SID)],
            out.at[pl.ds(t * NPAD + base_c + sid * ROWS_PER_SID,
                         ROWS_PER_SID)])
        plsc.subcore_barrier()


def _sc_agg(xs0, xs1, xs2, xs3, src, dst):
    f = functools.partial(
        pl.kernel,
        out_type=jax.ShapeDtypeStruct((4 * NPAD, H), jnp.float32),
        mesh=_sc_mesh,
        scratch_types=[
            pltpu.VMEM_SHARED((ACC_ROWS, H), jnp.float32),
            pltpu.VMEM((NJ, SUBV), jnp.int32),
            pltpu.VMEM((NJ, SUBV), jnp.int32),
            pltpu.VMEM((NJ, SUBV), jnp.int32),
            pltpu.VMEM((SUBV, H), jnp.float32),
            pltpu.VMEM((ZROWS, H), jnp.float32),
            pltpu.SemaphoreType.DMA,
        ],
    )(_sc_agg_body)
    return f(xs0, xs1, xs2, xs3, src, dst)


def _pack_edges(eis):
    pad = EPT - E
    srcs = [jnp.concatenate([ei[0], jnp.zeros((pad,), jnp.int32)])
            for ei in eis]
    dsts = [jnp.concatenate([ei[1], jnp.full((pad,), DUMMY_DST, jnp.int32)])
            for ei in eis]
    src = jnp.stack(srcs).reshape(4 * NSUB * NCHUNK * NJ, SUBV)
    dst = jnp.stack(dsts).reshape(4 * NSUB * NCHUNK * NJ, SUBV)
    return src, dst


def kernel(x_op, x_m, ei_op_op, ei_op_m, ei_m_op, ei_m_m, W_mtrans, b_mtrans,
           W_0_nn1_0, b_0_nn1_0, W_0_nn1_1, b_0_nn1_1,
           W_0_nn2_0, b_0_nn2_0, W_0_nn2_1, b_0_nn2_1,
           W_0_nn3_0, b_0_nn3_0, W_0_nn3_1, b_0_nn3_1,
           W_0_nn4_0, b_0_nn4_0, W_0_nn4_1, b_0_nn4_1,
           W_1_nn1_0, b_1_nn1_0, W_1_nn1_1, b_1_nn1_1,
           W_1_nn2_0, b_1_nn2_0, W_1_nn2_1, b_1_nn2_1,
           W_1_nn3_0, b_1_nn3_0, W_1_nn3_1, b_1_nn3_1,
           W_1_nn4_0, b_1_nn4_0, W_1_nn4_1, b_1_nn4_1,
           W_op_fc, b_op_fc, W_m_fc, b_m_fc):
    x_op_pad = jnp.pad(x_op, ((0, NPAD - N), (0, 16 - 7)))
    x_m_pad = jnp.pad(x_m, ((0, NPAD - N), (0, 8 - 4)))
    Wm_p = jnp.pad(W_mtrans, ((0, 8 - 4), (0, 16 - 7)))
    bm_p = jnp.pad(b_mtrans, (0, 16 - 7)).reshape(1, 16)

    def pad_w0(w):
        return jnp.pad(w, ((0, 9), (0, 0)))

    def row(b):
        return b.reshape(1, H)

    eis = (ei_op_op, ei_m_op, ei_op_m, ei_m_m)
    src, dst = _pack_edges(eis)

    y1, y2, y3, y4 = _pretrans(x_op_pad, x_m_pad, Wm_p, bm_p,
                               pad_w0(W_0_nn1_0), pad_w0(W_0_nn2_0),
                               pad_w0(W_0_nn3_0), pad_w0(W_0_nn4_0))
    agg0 = _sc_agg(y1, y3, y2, y4, src, dst)
    x_op1 = _layer0_half(agg0, 0, 1,
                         row(b_0_nn1_0), W_0_nn1_1, row(b_0_nn1_1),
                         row(b_0_nn3_0), W_0_nn3_1, row(b_0_nn3_1))
    x_m1 = _layer0_half(agg0, 2, 3,
                        row(b_0_nn2_0), W_0_nn2_1, row(b_0_nn2_1),
                        row(b_0_nn4_0), W_0_nn4_1, row(b_0_nn4_1))

    agg1 = _sc_agg(x_op1, x_m1, x_op1, x_m1, src, dst)
    out_op = _layer1_half(agg1, 0, 1,
                          W_1_nn1_0, row(b_1_nn1_0), W_1_nn1_1, row(b_1_nn1_1),
                          W_1_nn3_0, row(b_1_nn3_0), W_1_nn3_1, row(b_1_nn3_1),
                          W_op_fc, row(b_op_fc))
    out_m = _layer1_half(agg1, 2, 3,
                         W_1_nn2_0, row(b_1_nn2_0), W_1_nn2_1, row(b_1_nn2_1),
                         W_1_nn4_0, row(b_1_nn4_0), W_1_nn4_1, row(b_1_nn4_1),
                         W_m_fc, row(b_m_fc))
    return out_op, out_m


# speedup vs baseline: 1.0023x; 1.0023x over previous
import functools

import jax
import jax.numpy as jnp
from jax import lax
from jax.experimental import pallas as pl
from jax.experimental.pallas import tpu as pltpu
from jax.experimental.pallas import tpu_sc as plsc

N = 25000
NPAD = 25600
E = 150000
H = 128
R = 200
NB = NPAD // R
NBO = N // R
DUMMY_DST = N

NSUB = 16
SUBV = 128
NJ = 8
G = NJ * SUBV
NCHUNK = 10
EPT = NSUB * NCHUNK * G
HALF = NPAD // 2
ACC_ROWS = HALF + 8
ROWS_PER_SID = HALF // NSUB
ZROWS = 32
SLAB = 64
NSLAB = G // SLAB
QS = SLAB // 16

_sc_mesh = plsc.VectorSubcoreMesh(core_axis_name="c", subcore_axis_name="s")


def _leaky(x):
    return jnp.where(x >= 0, x, 0.01 * x)


def _pretrans_body(xop_ref, xm_ref, wm_ref, bm_ref,
                   w1_ref, w2_ref, w3_ref, w4_ref,
                   y1_ref, y2_ref, y3_ref, y4_ref):
    xop = xop_ref[...]
    y1_ref[...] = jnp.dot(xop, w1_ref[...], preferred_element_type=jnp.float32)
    y2_ref[...] = jnp.dot(xop, w2_ref[...], preferred_element_type=jnp.float32)
    xm7 = jnp.dot(xm_ref[...], wm_ref[...],
                  preferred_element_type=jnp.float32) + bm_ref[...]
    y3_ref[...] = jnp.dot(xm7, w3_ref[...], preferred_element_type=jnp.float32)
    y4_ref[...] = jnp.dot(xm7, w4_ref[...], preferred_element_type=jnp.float32)


def _pretrans(x_op_pad, x_m_pad, Wm_p, bm_p, w1, w2, w3, w4):
    out = jax.ShapeDtypeStruct((NPAD, H), jnp.float32)
    full = lambda shape: pl.BlockSpec(shape, lambda i: (0, 0))
    return pl.pallas_call(
        _pretrans_body,
        grid=(NB,),
        in_specs=[
            pl.BlockSpec((R, 16), lambda i: (i, 0)),
            pl.BlockSpec((R, 8), lambda i: (i, 0)),
            full((8, 16)), full((1, 16)),
            full((16, H)), full((16, H)), full((16, H)), full((16, H)),
        ],
        out_specs=[pl.BlockSpec((R, H), lambda i: (i, 0))] * 4,
        out_shape=[out, out, out, out],
    )(x_op_pad, x_m_pad, Wm_p, bm_p, w1, w2, w3, w4)


def _layer0_body(a1_ref, a3_ref, b1a_ref, w1b_ref, b1b_ref,
                 b3a_ref, w3b_ref, b3b_ref, out_ref):
    h1 = jnp.maximum(a1_ref[...] + b1a_ref[...], 0.0)
    t1 = jnp.dot(h1, w1b_ref[...], preferred_element_type=jnp.float32) + b1b_ref[...]
    h3 = jnp.maximum(a3_ref[...] + b3a_ref[...], 0.0)
    t3 = jnp.dot(h3, w3b_ref[...], preferred_element_type=jnp.float32) + b3b_ref[...]
    out_ref[...] = _leaky(t1 + t3)


def _layer0_half(agg0, t1, t3, b1a, w1b, b1b, b3a, w3b, b3b):
    full = lambda shape: pl.BlockSpec(shape, lambda i: (0, 0))
    return pl.pallas_call(
        _layer0_body,
        grid=(NB,),
        in_specs=[
            pl.BlockSpec((R, H), lambda i, t=t1: (i + t * NB, 0)),
            pl.BlockSpec((R, H), lambda i, t=t3: (i + t * NB, 0)),
            full((1, H)), full((H, H)), full((1, H)),
            full((1, H)), full((H, H)), full((1, H)),
        ],
        out_specs=pl.BlockSpec((R, H), lambda i: (i, 0)),
        out_shape=jax.ShapeDtypeStruct((NPAD, H), jnp.float32),
    )(agg0, agg0, b1a, w1b, b1b, b3a, w3b, b3b)


def _layer1_body(a1_ref, a3_ref,
                 w1a_ref, b1a_ref, w1b_ref, b1b_ref,
                 w3a_ref, b3a_ref, w3b_ref, b3b_ref,
                 wfc_ref, bfc_ref, out_ref):
    h1 = jnp.maximum(jnp.dot(a1_ref[...], w1a_ref[...],
                             preferred_element_type=jnp.float32) + b1a_ref[...], 0.0)
    t1 = jnp.dot(h1, w1b_ref[...], preferred_element_type=jnp.float32) + b1b_ref[...]
    h3 = jnp.maximum(jnp.dot(a3_ref[...], w3a_ref[...],
                             preferred_element_type=jnp.float32) + b3a_ref[...], 0.0)
    t3 = jnp.dot(h3, w3b_ref[...], preferred_element_type=jnp.float32) + b3b_ref[...]
    x = _leaky(t1 + t3)
    out_ref[...] = jnp.dot(x, wfc_ref[...],
                           preferred_element_type=jnp.float32) + bfc_ref[...]


def _layer1_half(agg1, t1, t3, w1a, b1a, w1b, b1b, w3a, b3a, w3b, b3b,
                 wfc, bfc):
    full = lambda shape: pl.BlockSpec(shape, lambda i: (0, 0))
    return pl.pallas_call(
        _layer1_body,
        grid=(NBO,),
        in_specs=[
            pl.BlockSpec((R, H), lambda i, t=t1: (i + t * NB, 0)),
            pl.BlockSpec((R, H), lambda i, t=t3: (i + t * NB, 0)),
            full((H, H)), full((1, H)), full((H, H)), full((1, H)),
            full((H, H)), full((1, H)), full((H, H)), full((1, H)),
            full((H, H)), full((1, H)),
        ],
        out_specs=pl.BlockSpec((R, H), lambda i: (i, 0)),
        out_shape=jax.ShapeDtypeStruct((N, H), jnp.float32),
    )(agg1, agg1, w1a, b1a, w1b, b1b, w3a, b3a, w3b, b3b, wfc, bfc)


def _sc_agg_body(xs0, xs1, xs2, xs3, src, dst, out,
                 acc, sidx, didx, gbuf0, gbuf1, zbuf,
                 gsem0, gsem1, ssem0, ssem1, isem):
    c = lax.axis_index("c")
    sid = lax.axis_index("s")
    base_c = c * HALF
    gbufs = (gbuf0, gbuf1)
    gsems = (gsem0, gsem1)
    ssems = (ssem0, ssem1)

    z16 = jnp.zeros((16,), jnp.float32)

    @pl.loop(0, ZROWS)
    def _(i):
        for q in range(H // 16):
            zbuf[i, pl.ds(q * 16, 16)] = z16

    for t, xsrc in enumerate((xs0, xs1, xs2, xs3)):
        @pl.loop(0, ROWS_PER_SID // ZROWS)
        def _(j):
            pltpu.sync_copy(
                zbuf, acc.at[pl.ds(sid * ROWS_PER_SID + j * ZROWS, ZROWS)])
        plsc.subcore_barrier()

        @pl.loop(0, NCHUNK)
        def _(k, xsrc=xsrc, t=t):
            row0 = ((t * NSUB + sid) * NCHUNK + k) * NJ
            pltpu.async_copy(src.at[pl.ds(row0, NJ)], sidx, isem).wait()
            pltpu.async_copy(dst.at[pl.ds(row0, NJ)], didx, isem).wait()
            for j in range(NJ):
                for q in range(SUBV // 16):
                    d = didx[j, pl.ds(q * 16, 16)]
                    dl = d - base_c
                    ok = (dl >= 0) & (dl < HALF)
                    didx[j, pl.ds(q * 16, 16)] = jnp.where(
                        ok, dl, HALF + (d & 7))

            def gather(s):
                b = s % 2
                j, h = divmod(s, 2)
                return pltpu.async_copy(
                    xsrc.at[sidx.at[j, pl.ds(h * SLAB, SLAB)]],
                    gbufs[b], gsems[b])

            def scatter(s):
                b = s % 2
                j, h = divmod(s, 2)
                hs = []
                for q in range(QS):
                    dv = didx[j, pl.ds(h * SLAB + q * 16, 16)]
                    hs.append(pltpu.async_copy(
                        gbufs[b].at[pl.ds(q * 16, 16)],
                        acc.at[dv], ssems[b], add=True))
                return hs

            pend = {}
            g = {0: gather(0)}
            for s in range(1, NSLAB):
                b = s % 2
                if s >= 2:
                    for hnd in pend.pop(s - 2):
                        hnd.wait()
                g[s] = gather(s)
                g.pop(s - 1).wait()
                pend[s - 1] = scatter(s - 1)
            for hnd in pend.pop(NSLAB - 3, []):
                hnd.wait()
            g.pop(NSLAB - 1).wait()
            for hnd in scatter(NSLAB - 1):
                hnd.wait()
            for hnd in pend.pop(NSLAB - 2):
                hnd.wait()

        plsc.subcore_barrier()
        pltpu.sync_copy(
            acc.at[pl.ds(sid * ROWS_PER_SID, ROWS_PER_SID)],
            out.at[pl.ds(t * NPAD + base_c + sid * ROWS_PER_SID,
                         ROWS_PER_SID)])
        plsc.subcore_barrier()


def _sc_agg(xs0, xs1, xs2, xs3, src, dst):
    f = functools.partial(
        pl.kernel,
        out_type=jax.ShapeDtypeStruct((4 * NPAD, H), jnp.float32),
        mesh=_sc_mesh,
        scratch_types=[
            pltpu.VMEM_SHARED((ACC_ROWS, H), jnp.float32),
            pltpu.VMEM((NJ, SUBV), jnp.int32),
            pltpu.VMEM((NJ, SUBV), jnp.int32),
            pltpu.VMEM((SLAB, H), jnp.float32),
            pltpu.VMEM((SLAB, H), jnp.float32),
            pltpu.VMEM((ZROWS, H), jnp.float32),
            pltpu.SemaphoreType.DMA,
            pltpu.SemaphoreType.DMA,
            pltpu.SemaphoreType.DMA,
            pltpu.SemaphoreType.DMA,
            pltpu.SemaphoreType.DMA,
        ],
    )(_sc_agg_body)
    return f(xs0, xs1, xs2, xs3, src, dst)


def _pack_edges(eis):
    pad = EPT - E
    srcs = [jnp.concatenate([ei[0], jnp.zeros((pad,), jnp.int32)])
            for ei in eis]
    dsts = [jnp.concatenate([ei[1], jnp.full((pad,), DUMMY_DST, jnp.int32)])
            for ei in eis]
    src = jnp.stack(srcs).reshape(4 * NSUB * NCHUNK * NJ, SUBV)
    dst = jnp.stack(dsts).reshape(4 * NSUB * NCHUNK * NJ, SUBV)
    return src, dst


def kernel(x_op, x_m, ei_op_op, ei_op_m, ei_m_op, ei_m_m, W_mtrans, b_mtrans,
           W_0_nn1_0, b_0_nn1_0, W_0_nn1_1, b_0_nn1_1,
           W_0_nn2_0, b_0_nn2_0, W_0_nn2_1, b_0_nn2_1,
           W_0_nn3_0, b_0_nn3_0, W_0_nn3_1, b_0_nn3_1,
           W_0_nn4_0, b_0_nn4_0, W_0_nn4_1, b_0_nn4_1,
           W_1_nn1_0, b_1_nn1_0, W_1_nn1_1, b_1_nn1_1,
           W_1_nn2_0, b_1_nn2_0, W_1_nn2_1, b_1_nn2_1,
           W_1_nn3_0, b_1_nn3_0, W_1_nn3_1, b_1_nn3_1,
           W_1_nn4_0, b_1_nn4_0, W_1_nn4_1, b_1_nn4_1,
           W_op_fc, b_op_fc, W_m_fc, b_m_fc):
    x_op_pad = jnp.pad(x_op, ((0, NPAD - N), (0, 16 - 7)))
    x_m_pad = jnp.pad(x_m, ((0, NPAD - N), (0, 8 - 4)))
    Wm_p = jnp.pad(W_mtrans, ((0, 8 - 4), (0, 16 - 7)))
    bm_p = jnp.pad(b_mtrans, (0, 16 - 7)).reshape(1, 16)

    def pad_w0(w):
        return jnp.pad(w, ((0, 9), (0, 0)))

    def row(b):
        return b.reshape(1, H)

    eis = (ei_op_op, ei_m_op, ei_op_m, ei_m_m)
    src, dst = _pack_edges(eis)

    y1, y2, y3, y4 = _pretrans(x_op_pad, x_m_pad, Wm_p, bm_p,
                               pad_w0(W_0_nn1_0), pad_w0(W_0_nn2_0),
                               pad_w0(W_0_nn3_0), pad_w0(W_0_nn4_0))
    agg0 = _sc_agg(y1, y3, y2, y4, src, dst)
    x_op1 = _layer0_half(agg0, 0, 1,
                         row(b_0_nn1_0), W_0_nn1_1, row(b_0_nn1_1),
                         row(b_0_nn3_0), W_0_nn3_1, row(b_0_nn3_1))
    x_m1 = _layer0_half(agg0, 2, 3,
                        row(b_0_nn2_0), W_0_nn2_1, row(b_0_nn2_1),
                        row(b_0_nn4_0), W_0_nn4_1, row(b_0_nn4_1))

    agg1 = _sc_agg(x_op1, x_m1, x_op1, x_m1, src, dst)
    out_op = _layer1_half(agg1, 0, 1,
                          W_1_nn1_0, row(b_1_nn1_0), W_1_nn1_1, row(b_1_nn1_1),
                          W_1_nn3_0, row(b_1_nn3_0), W_1_nn3_1, row(b_1_nn3_1),
                          W_op_fc, row(b_op_fc))
    out_m = _layer1_half(agg1, 2, 3,
                         W_1_nn2_0, row(b_1_nn2_0), W_1_nn2_1, row(b_1_nn2_1),
                         W_1_nn4_0, row(b_1_nn4_0), W_1_nn4_1, row(b_1_nn4_1),
                         W_m_fc, row(b_m_fc))
    return out_op, out_m
